# baseline (speedup 1.0000x reference)
import jax
import jax.numpy as jnp
from jax import lax
from jax.experimental import pallas as pl
from jax.experimental.pallas import tpu as pltpu
from jax.experimental.pallas import tpu_sc as plsc

NUM_CORES = 2
NUM_SUBCORES = 16
NUM_WORKERS = NUM_CORES * NUM_SUBCORES
LANES = 16
D = 64
ROWS_PER_SUB = 62500
MAX_CHUNK = 625
B_PER_TILE = 128
SEQ = 50


def _fused_body(idx_hbm, table_hbm, out_hbm, part_hbm,
                idx_v, mbuf0, mbuf1, rows0, rows1, out0, out1,
                pex_v, dbl_v, acc_v,
                s0, s1, g0, g1, o0, o1):
    cid = lax.axis_index("c")
    sid = lax.axis_index("s")

    mbase = sid * ROWS_PER_SUB

    def mchunk_start(j, buf, sem):
        pltpu.async_copy(
            table_hbm.at[pl.ds(mbase + j * MAX_CHUNK, MAX_CHUNK)], buf, sem)

    def mchunk_wait(buf, sem):
        pltpu.make_async_copy(
            table_hbm.at[pl.ds(mbase, MAX_CHUNK)], buf, sem).wait()

    def mchunk_reduce(buf, acc):
        def row_body(r, a):
            for c in range(D // LANES):
                a = jnp.maximum(a, jnp.abs(buf[r, pl.ds(c * LANES, LANES)]))
            return a

        return lax.fori_loop(0, MAX_CHUNK, row_body, acc, unroll=4)

    mchunk_start(0, mbuf0, s0)
    mchunk_start(1, mbuf1, s1)
    n_mpairs = ROWS_PER_SUB // MAX_CHUNK // 2

    def mbody(t, acc):
        mchunk_wait(mbuf0, s0)
        acc = mchunk_reduce(mbuf0, acc)

        @pl.when(t < n_mpairs - 1)
        def _():
            mchunk_start(2 * t + 2, mbuf0, s0)

        mchunk_wait(mbuf1, s1)
        acc = mchunk_reduce(mbuf1, acc)

        @pl.when(t < n_mpairs - 1)
        def _():
            mchunk_start(2 * t + 3, mbuf1, s1)

        return acc

    acc = lax.fori_loop(0, n_mpairs, mbody, jnp.zeros((LANES,), jnp.float32))
    acc_v[...] = acc
    pltpu.sync_copy(acc_v, part_hbm.at[cid, sid])
    plsc.subcore_barrier()
    pltpu.sync_copy(part_hbm.at[cid], pex_v)

    m = pex_v[0, :]
    for i in range(1, NUM_SUBCORES):
        m = jnp.maximum(m, pex_v[i, :])
    dbl_v[pl.ds(0, LANES)] = m
    dbl_v[pl.ds(LANES, LANES)] = m
    vmax = m
    for k in range(1, LANES):
        vmax = jnp.maximum(vmax, dbl_v[pl.ds(k, LANES)])
    vneg = -vmax
    vthr = vmax * (2.0 ** -24)

    wid = sid * NUM_CORES + cid
    b0 = wid * B_PER_TILE
    pltpu.sync_copy(idx_hbm.at[wid], idx_v)

    def quantize(rows_v, out_v):
        def row_body(r, carry):
            for c in range(D // LANES):
                w = rows_v[r, pl.ds(c * LANES, LANES)]
                out_v[r, pl.ds(c * LANES, LANES)] = jnp.where(
                    w > vthr, vmax, vneg)
            return carry

        lax.fori_loop(0, SEQ, row_body, 0, unroll=2)

    pltpu.async_copy(table_hbm.at[idx_v.at[0]], rows0, g0)
    pltpu.async_copy(table_hbm.at[idx_v.at[1]], rows1, g1)
    n_pairs = B_PER_TILE // 2

    def body(t, carry):
        pltpu.make_async_copy(table_hbm.at[idx_v.at[2 * t]], rows0, g0).wait()

        @pl.when(t > 0)
        def _():
            pltpu.make_async_copy(out0, out_hbm.at[b0], o0).wait()

        quantize(rows0, out0)
        pltpu.async_copy(out0, out_hbm.at[b0 + 2 * t], o0)

        @pl.when(t < n_pairs - 1)
        def _():
            pltpu.async_copy(table_hbm.at[idx_v.at[2 * t + 2]], rows0, g0)

        pltpu.make_async_copy(
            table_hbm.at[idx_v.at[2 * t + 1]], rows1, g1).wait()

        @pl.when(t > 0)
        def _():
            pltpu.make_async_copy(out1, out_hbm.at[b0], o1).wait()

        quantize(rows1, out1)
        pltpu.async_copy(out1, out_hbm.at[b0 + 2 * t + 1], o1)

        @pl.when(t < n_pairs - 1)
        def _():
            pltpu.async_copy(table_hbm.at[idx_v.at[2 * t + 3]], rows1, g1)

        return carry

    lax.fori_loop(0, n_pairs, body, 0)
    pltpu.make_async_copy(out0, out_hbm.at[b0], o0).wait()
    pltpu.make_async_copy(out1, out_hbm.at[b0], o1).wait()


def _fused(idx3, weight):
    b = NUM_WORKERS * B_PER_TILE
    mesh = plsc.VectorSubcoreMesh(core_axis_name="c", subcore_axis_name="s")
    f = pl.kernel(
        _fused_body,
        out_type=(
            jax.ShapeDtypeStruct((b, SEQ, D), jnp.float32),
            jax.ShapeDtypeStruct((NUM_CORES, NUM_SUBCORES, LANES),
                                 jnp.float32),
        ),
        mesh=mesh,
        scratch_types=[
            pltpu.VMEM((B_PER_TILE, SEQ), jnp.int32),
            pltpu.VMEM((MAX_CHUNK, D), jnp.float32),
            pltpu.VMEM((MAX_CHUNK, D), jnp.float32),
            pltpu.VMEM((SEQ, D), jnp.float32),
            pltpu.VMEM((SEQ, D), jnp.float32),
            pltpu.VMEM((SEQ, D), jnp.float32),
            pltpu.VMEM((SEQ, D), jnp.float32),
            pltpu.VMEM((NUM_SUBCORES, LANES), jnp.float32),
            pltpu.VMEM((2 * LANES,), jnp.float32),
            pltpu.VMEM((LANES,), jnp.float32),
            pltpu.SemaphoreType.DMA,
            pltpu.SemaphoreType.DMA,
            pltpu.SemaphoreType.DMA,
            pltpu.SemaphoreType.DMA,
            pltpu.SemaphoreType.DMA,
            pltpu.SemaphoreType.DMA,
        ],
        compiler_params=pltpu.CompilerParams(use_tc_tiling_on_sc=False),
    )
    out, _ = f(idx3, weight)
    return out


def kernel(input, weight):
    b, s = input.shape
    assert b == NUM_WORKERS * B_PER_TILE and s == SEQ
    idx3 = input.astype(jnp.int32).reshape(NUM_WORKERS, B_PER_TILE, SEQ)
    return _fused(idx3, weight)


# speedup vs baseline: 1.0957x; 1.0957x over previous
import jax
import jax.numpy as jnp
from jax import lax
from jax.experimental import pallas as pl
from jax.experimental.pallas import tpu as pltpu
from jax.experimental.pallas import tpu_sc as plsc

NUM_CORES = 2
NUM_SUBCORES = 16
NUM_WORKERS = NUM_CORES * NUM_SUBCORES
LANES = 16
D = 64
ROWS_PER_TILE = 31250
MAX_CHUNK = 625
CHUNK = 128
N_CHUNKS = 50


def _wid():
    return lax.axis_index("s") * NUM_CORES + lax.axis_index("c")


def _max_body(table_hbm, part_hbm, buf0, buf1, acc_v, s0, s1):
    wid = _wid()
    base = wid * ROWS_PER_TILE

    def chunk_start(j, buf, sem):
        pltpu.async_copy(
            table_hbm.at[pl.ds(base + j * MAX_CHUNK, MAX_CHUNK)], buf, sem)

    def chunk_wait(buf, sem):
        pltpu.make_async_copy(
            table_hbm.at[pl.ds(base, MAX_CHUNK)], buf, sem).wait()

    def chunk_reduce(buf, acc):
        def row_body(r, a):
            for c in range(D // LANES):
                a = jnp.maximum(a, jnp.abs(buf[r, pl.ds(c * LANES, LANES)]))
            return a

        return lax.fori_loop(0, MAX_CHUNK, row_body, acc, unroll=4)

    chunk_start(0, buf0, s0)
    chunk_start(1, buf1, s1)
    n_pairs = ROWS_PER_TILE // MAX_CHUNK // 2

    def body(t, acc):
        chunk_wait(buf0, s0)
        acc = chunk_reduce(buf0, acc)

        @pl.when(t < n_pairs - 1)
        def _():
            chunk_start(2 * t + 2, buf0, s0)

        chunk_wait(buf1, s1)
        acc = chunk_reduce(buf1, acc)

        @pl.when(t < n_pairs - 1)
        def _():
            chunk_start(2 * t + 3, buf1, s1)

        return acc

    acc = lax.fori_loop(0, n_pairs, body, jnp.zeros((LANES,), jnp.float32))
    acc_v[...] = acc
    pltpu.sync_copy(acc_v, part_hbm.at[wid])


def _max_partials(weight):
    mesh = plsc.VectorSubcoreMesh(core_axis_name="c", subcore_axis_name="s")
    f = pl.kernel(
        _max_body,
        out_type=jax.ShapeDtypeStruct((NUM_WORKERS, LANES), jnp.float32),
        mesh=mesh,
        scratch_types=[
            pltpu.VMEM((MAX_CHUNK, D), jnp.float32),
            pltpu.VMEM((MAX_CHUNK, D), jnp.float32),
            pltpu.VMEM((LANES,), jnp.float32),
            pltpu.SemaphoreType.DMA,
            pltpu.SemaphoreType.DMA,
        ],
        compiler_params=pltpu.CompilerParams(use_tc_tiling_on_sc=False),
    )
    return f(weight)


def _gather_body(idx_hbm, table_hbm, maxv_hbm, out_hbm,
                 idx_v, maxv_v, rows0, rows1, out0, out1,
                 g0, g1, o0, o1):
    wid = _wid()
    base = wid * (N_CHUNKS * CHUNK)

    pltpu.sync_copy(idx_hbm.at[wid], idx_v)
    pltpu.sync_copy(maxv_hbm, maxv_v)

    vmax = maxv_v[...]
    vneg = -vmax
    vthr = vmax * (2.0 ** -24)

    def quantize(rows_v, out_v):
        def row_body(r, carry):
            for c in range(D // LANES):
                w = rows_v[r, pl.ds(c * LANES, LANES)]
                out_v[r, pl.ds(c * LANES, LANES)] = jnp.where(
                    w > vthr, vmax, vneg)
            return carry

        lax.fori_loop(0, CHUNK, row_body, 0, unroll=4)

    pltpu.async_copy(table_hbm.at[idx_v.at[0]], rows0, g0)
    pltpu.async_copy(table_hbm.at[idx_v.at[1]], rows1, g1)
    n_pairs = N_CHUNKS // 2

    def out_slice(j):
        return out_hbm.at[pl.ds(base + j * CHUNK, CHUNK)]

    def body(t, carry):
        pltpu.make_async_copy(table_hbm.at[idx_v.at[2 * t]], rows0, g0).wait()

        @pl.when(t > 0)
        def _():
            pltpu.make_async_copy(out0, out_slice(0), o0).wait()

        quantize(rows0, out0)
        pltpu.async_copy(out0, out_slice(2 * t), o0)

        @pl.when(t < n_pairs - 1)
        def _():
            pltpu.async_copy(table_hbm.at[idx_v.at[2 * t + 2]], rows0, g0)

        pltpu.make_async_copy(
            table_hbm.at[idx_v.at[2 * t + 1]], rows1, g1).wait()

        @pl.when(t > 0)
        def _():
            pltpu.make_async_copy(out1, out_slice(0), o1).wait()

        quantize(rows1, out1)
        pltpu.async_copy(out1, out_slice(2 * t + 1), o1)

        @pl.when(t < n_pairs - 1)
        def _():
            pltpu.async_copy(table_hbm.at[idx_v.at[2 * t + 3]], rows1, g1)

        return carry

    lax.fori_loop(0, n_pairs, body, 0)
    pltpu.make_async_copy(out0, out_slice(0), o0).wait()
    pltpu.make_async_copy(out1, out_slice(0), o1).wait()


def _gather_quant(idx3, weight, maxvec):
    total = NUM_WORKERS * N_CHUNKS * CHUNK
    mesh = plsc.VectorSubcoreMesh(core_axis_name="c", subcore_axis_name="s")
    f = pl.kernel(
        _gather_body,
        out_type=jax.ShapeDtypeStruct((total, D), jnp.float32),
        mesh=mesh,
        scratch_types=[
            pltpu.VMEM((N_CHUNKS, CHUNK), jnp.int32),
            pltpu.VMEM((LANES,), jnp.float32),
            pltpu.VMEM((CHUNK, D), jnp.float32),
            pltpu.VMEM((CHUNK, D), jnp.float32),
            pltpu.VMEM((CHUNK, D), jnp.float32),
            pltpu.VMEM((CHUNK, D), jnp.float32),
            pltpu.SemaphoreType.DMA,
            pltpu.SemaphoreType.DMA,
            pltpu.SemaphoreType.DMA,
            pltpu.SemaphoreType.DMA,
        ],
        compiler_params=pltpu.CompilerParams(use_tc_tiling_on_sc=False),
    )
    return f(idx3, weight, maxvec)


def kernel(input, weight):
    b, s = input.shape
    total = b * s
    assert NUM_WORKERS * N_CHUNKS * CHUNK == total
    idx3 = input.astype(jnp.int32).reshape(NUM_WORKERS, N_CHUNKS, CHUNK)
    partials = _max_partials(weight)
    maxvec = jnp.broadcast_to(jnp.max(partials), (LANES,))
    out = _gather_quant(idx3, weight, maxvec)
    return out.reshape(b, s, D)
